# baseline (device time: 22196 ns/iter reference)
import jax
import jax.numpy as jnp
from jax import lax
from jax.experimental import pallas as pl
from jax.experimental.pallas import tpu as pltpu

C = 32
LAG = 8


def kernel(x):
    m, n = x.shape
    half = m // 2
    rows = half // C

    def body(x_ref, out_ref, send_buf, comm_x, comm_y,
             x_send, x_recv, y_send, y_recv):
        my_x = lax.axis_index("x")
        my_y = lax.axis_index("y")
        my_z = lax.axis_index("z")
        peer_x = (1 - my_x, my_y, my_z)
        peer_y = (my_x, 1 - my_y, my_z)

        x_off = my_y * half
        y_off = (1 - my_y) * half

        barrier = pltpu.get_barrier_semaphore()
        for nbr in (peer_x, peer_y):
            pl.semaphore_signal(
                barrier, inc=1, device_id=nbr,
                device_id_type=pl.DeviceIdType.MESH,
            )
        pl.semaphore_wait(barrier, 2)

        x_rdmas = []
        for c in range(C):
            send_buf[pl.ds(c * rows, rows), :] = (
                x_ref[pl.ds(x_off + c * rows, rows), :].astype(jnp.bfloat16)
            )
            r = pltpu.make_async_remote_copy(
                src_ref=send_buf.at[pl.ds(c * rows, rows), :],
                dst_ref=comm_x.at[c],
                send_sem=x_send.at[c],
                recv_sem=x_recv.at[c],
                device_id=peer_x,
                device_id_type=pl.DeviceIdType.MESH,
            )
            r.start()
            x_rdmas.append(r)

        def fold_y(c):
            y_rdmas[c].wait_recv()
            out_ref[pl.ds(y_off + c * rows, rows), :] = (
                x_ref[pl.ds(y_off + c * rows, rows), :]
                + comm_y[c, :, :].astype(jnp.float32)
            ).astype(jnp.bfloat16)

        y_rdmas = []
        for c in range(C):
            x_rdmas[c].wait_recv()
            r = pltpu.make_async_remote_copy(
                src_ref=comm_x.at[c],
                dst_ref=comm_y.at[c],
                send_sem=y_send.at[c],
                recv_sem=y_recv.at[c],
                device_id=peer_y,
                device_id_type=pl.DeviceIdType.MESH,
            )
            r.start()
            y_rdmas.append(r)
            out_ref[pl.ds(x_off + c * rows, rows), :] = (
                send_buf[pl.ds(c * rows, rows), :] + comm_x[c, :, :]
            )
            if c >= LAG:
                fold_y(c - LAG)

        for c in range(C - LAG, C):
            fold_y(c)

        for c in range(C):
            x_rdmas[c].wait_send()
            y_rdmas[c].wait_send()

    return pl.pallas_call(
        body,
        out_shape=jax.ShapeDtypeStruct((m, n), jnp.bfloat16),
        in_specs=[pl.BlockSpec(memory_space=pltpu.VMEM)],
        out_specs=pl.BlockSpec(memory_space=pltpu.VMEM),
        scratch_shapes=[
            pltpu.VMEM((half, n), jnp.bfloat16),
            pltpu.VMEM((C, rows, n), jnp.bfloat16),
            pltpu.VMEM((C, rows, n), jnp.bfloat16),
            pltpu.SemaphoreType.DMA((C,)),
            pltpu.SemaphoreType.DMA((C,)),
            pltpu.SemaphoreType.DMA((C,)),
            pltpu.SemaphoreType.DMA((C,)),
        ],
        compiler_params=pltpu.CompilerParams(collective_id=0),
    )(x)


# device time: 20269 ns/iter; 1.0951x vs baseline; 1.0951x over previous
import jax
import jax.numpy as jnp
from jax import lax
from jax.experimental import pallas as pl
from jax.experimental.pallas import tpu as pltpu

C = 8
H = C // 2


def kernel(x):
    m, n = x.shape
    quarter = m // 4
    rows = quarter // C

    def body(x_ref, out_ref, send_buf, comm_x, comm_y, comm_z, comm_d,
             x_s, x_r, yf_s, yf_r, zf_s, zf_r, yr_s, yr_r, zr_s, zr_r):
        my_x = lax.axis_index("x")
        my_y = lax.axis_index("y")
        my_z = lax.axis_index("z")
        p = lax.rem(my_z, 2)
        peer_x = (1 - my_x, my_y, my_z)
        peer_y = (my_x, 1 - my_y, my_z)
        peer_z = (my_x, my_y, my_z + 1 - 2 * p)

        off_m = (2 * p + my_y) * quarter
        off_y = (2 * p + (1 - my_y)) * quarter
        off_z = (2 * (1 - p) + my_y) * quarter
        off_d = (2 * (1 - p) + (1 - my_y)) * quarter

        barrier = pltpu.get_barrier_semaphore()
        for nbr in (peer_x, peer_y, peer_z):
            pl.semaphore_signal(
                barrier, inc=1, device_id=nbr,
                device_id_type=pl.DeviceIdType.MESH,
            )
        pl.semaphore_wait(barrier, 3)

        x_rd = []
        for c in range(C):
            send_buf[pl.ds(c * rows, rows), :] = (
                x_ref[pl.ds(off_m + c * rows, rows), :].astype(jnp.bfloat16)
            )
            r = pltpu.make_async_remote_copy(
                src_ref=send_buf.at[pl.ds(c * rows, rows), :],
                dst_ref=comm_x.at[c],
                send_sem=x_s.at[c],
                recv_sem=x_r.at[c],
                device_id=peer_x,
                device_id_type=pl.DeviceIdType.MESH,
            )
            r.start()
            x_rd.append(r)

        def fwd(src, dst, ssem, csem, c, peer):
            r = pltpu.make_async_remote_copy(
                src_ref=src.at[c], dst_ref=dst.at[c],
                send_sem=ssem.at[c], recv_sem=csem.at[c],
                device_id=peer, device_id_type=pl.DeviceIdType.MESH,
            )
            r.start()
            return r

        def fold(off, c, buf):
            out_ref[pl.ds(off + c * rows, rows), :] = (
                x_ref[pl.ds(off + c * rows, rows), :]
                + buf[c, :, :].astype(jnp.float32)
            ).astype(jnp.bfloat16)

        yf_rd, zf_rd = [], []
        for c in range(C):
            x_rd[c].wait_recv()
            yf_rd.append(fwd(comm_x, comm_y, yf_s, yf_r, c, peer_y))
            zf_rd.append(fwd(comm_x, comm_z, zf_s, zf_r, c, peer_z))
            out_ref[pl.ds(off_m + c * rows, rows), :] = (
                send_buf[pl.ds(c * rows, rows), :] + comm_x[c, :, :]
            )

        yr_rd, zr_rd = [], []
        for c in range(H):
            zf_rd[c].wait_recv()
            r = pltpu.make_async_remote_copy(
                src_ref=comm_z.at[c], dst_ref=comm_d.at[c],
                send_sem=yr_s.at[c], recv_sem=yr_r.at[c],
                device_id=peer_y, device_id_type=pl.DeviceIdType.MESH,
            )
            r.start()
            yr_rd.append(r)
            fold(off_z, c, comm_z)
            yf_rd[H + c].wait_recv()
            r = pltpu.make_async_remote_copy(
                src_ref=comm_y.at[H + c], dst_ref=comm_d.at[H + c],
                send_sem=zr_s.at[c], recv_sem=zr_r.at[c],
                device_id=peer_z, device_id_type=pl.DeviceIdType.MESH,
            )
            r.start()
            zr_rd.append(r)
            fold(off_y, H + c, comm_y)

        for c in range(H):
            yf_rd[c].wait_recv()
            fold(off_y, c, comm_y)
            zf_rd[H + c].wait_recv()
            fold(off_z, H + c, comm_z)

        for c in range(H):
            yr_rd[c].wait_recv()
            fold(off_d, c, comm_d)
        for c in range(H):
            zr_rd[c].wait_recv()
            fold(off_d, H + c, comm_d)

        for r in x_rd + yf_rd + zf_rd + yr_rd + zr_rd:
            r.wait_send()

    return pl.pallas_call(
        body,
        out_shape=jax.ShapeDtypeStruct((m, n), jnp.bfloat16),
        in_specs=[pl.BlockSpec(memory_space=pltpu.VMEM)],
        out_specs=pl.BlockSpec(memory_space=pltpu.VMEM),
        scratch_shapes=[
            pltpu.VMEM((quarter, n), jnp.bfloat16),
            pltpu.VMEM((C, rows, n), jnp.bfloat16),
            pltpu.VMEM((C, rows, n), jnp.bfloat16),
            pltpu.VMEM((C, rows, n), jnp.bfloat16),
            pltpu.VMEM((C, rows, n), jnp.bfloat16),
            pltpu.SemaphoreType.DMA((C,)),
            pltpu.SemaphoreType.DMA((C,)),
            pltpu.SemaphoreType.DMA((C,)),
            pltpu.SemaphoreType.DMA((C,)),
            pltpu.SemaphoreType.DMA((C,)),
            pltpu.SemaphoreType.DMA((C,)),
            pltpu.SemaphoreType.DMA((H,)),
            pltpu.SemaphoreType.DMA((H,)),
            pltpu.SemaphoreType.DMA((H,)),
            pltpu.SemaphoreType.DMA((H,)),
        ],
        compiler_params=pltpu.CompilerParams(collective_id=0),
    )(x)


# device time: 18736 ns/iter; 1.1847x vs baseline; 1.0818x over previous
import jax
import jax.numpy as jnp
from jax import lax
from jax.experimental import pallas as pl
from jax.experimental.pallas import tpu as pltpu

CQ = 16
K2 = 2
K1 = 6
NF = CQ - K2
ND = CQ - K1
HD = ND // 2
NX = CQ + 2 * K2 + K1


def kernel(x):
    m, n = x.shape
    quarter = m // 4
    rows = quarter // CQ

    def body(x_ref, out_ref, send_buf, comm_x, comm_y, comm_z, comm_d,
             x_s, x_r, yf_s, yf_r, zf_s, zf_r, yr_s, yr_r, zr_s, zr_r):
        my_x = lax.axis_index("x")
        my_y = lax.axis_index("y")
        my_z = lax.axis_index("z")
        p = lax.rem(my_z, 2)
        peer_x = (1 - my_x, my_y, my_z)
        peer_y = (my_x, 1 - my_y, my_z)
        peer_z = (my_x, my_y, my_z + 1 - 2 * p)

        off_m = (2 * p + my_y) * quarter
        off_y = (2 * p + (1 - my_y)) * quarter
        off_z = (2 * (1 - p) + my_y) * quarter
        off_d = (2 * (1 - p) + (1 - my_y)) * quarter

        slots = (
            [(c, off_m, c) for c in range(CQ)]
            + [(CQ + i, off_y, NF + i) for i in range(K2)]
            + [(CQ + K2 + i, off_z, NF + i) for i in range(K2)]
            + [(CQ + 2 * K2 + i, off_d, ND + i) for i in range(K1)]
        )

        barrier = pltpu.get_barrier_semaphore()
        for nbr in (peer_x, peer_y, peer_z):
            pl.semaphore_signal(
                barrier, inc=1, device_id=nbr,
                device_id_type=pl.DeviceIdType.MESH,
            )
        pl.semaphore_wait(barrier, 3)

        x_rd = []
        for slot, off, c in slots:
            send_buf[slot, :, :] = (
                x_ref[pl.ds(off + c * rows, rows), :].astype(jnp.bfloat16)
            )
            r = pltpu.make_async_remote_copy(
                src_ref=send_buf.at[slot],
                dst_ref=comm_x.at[slot],
                send_sem=x_s.at[slot],
                recv_sem=x_r.at[slot],
                device_id=peer_x,
                device_id_type=pl.DeviceIdType.MESH,
            )
            r.start()
            x_rd.append(r)

        def fold(off, c, buf, slot):
            out_ref[pl.ds(off + c * rows, rows), :] = (
                x_ref[pl.ds(off + c * rows, rows), :]
                + buf[slot, :, :].astype(jnp.float32)
            ).astype(jnp.bfloat16)

        yf_rd, zf_rd = [], []
        for c in range(NF):
            x_rd[c].wait_recv()
            for dst, ss, rs, peer, lst in (
                (comm_y, yf_s, yf_r, peer_y, yf_rd),
                (comm_z, zf_s, zf_r, peer_z, zf_rd),
            ):
                r = pltpu.make_async_remote_copy(
                    src_ref=comm_x.at[c], dst_ref=dst.at[c],
                    send_sem=ss.at[c], recv_sem=rs.at[c],
                    device_id=peer, device_id_type=pl.DeviceIdType.MESH,
                )
                r.start()
                lst.append(r)
            out_ref[pl.ds(off_m + c * rows, rows), :] = (
                send_buf[c, :, :] + comm_x[c, :, :]
            )

        yr_rd, zr_rd = [], []
        for c in range(HD):
            zf_rd[c].wait_recv()
            r = pltpu.make_async_remote_copy(
                src_ref=comm_z.at[c], dst_ref=comm_d.at[c],
                send_sem=yr_s.at[c], recv_sem=yr_r.at[c],
                device_id=peer_y, device_id_type=pl.DeviceIdType.MESH,
            )
            r.start()
            yr_rd.append(r)
            fold(off_z, c, comm_z, c)
            yf_rd[HD + c].wait_recv()
            r = pltpu.make_async_remote_copy(
                src_ref=comm_y.at[HD + c], dst_ref=comm_d.at[HD + c],
                send_sem=zr_s.at[c], recv_sem=zr_r.at[c],
                device_id=peer_z, device_id_type=pl.DeviceIdType.MESH,
            )
            r.start()
            zr_rd.append(r)
            fold(off_y, HD + c, comm_y, HD + c)

        for c in range(HD):
            yf_rd[c].wait_recv()
            fold(off_y, c, comm_y, c)
        for c in range(2 * HD, NF):
            yf_rd[c].wait_recv()
            fold(off_y, c, comm_y, c)
        for c in range(HD, NF):
            zf_rd[c].wait_recv()
            fold(off_z, c, comm_z, c)

        for i, (slot, off, c) in enumerate(slots[CQ:]):
            x_rd[CQ + i].wait_recv()
            fold(off, c, comm_x, slot)
        for c in range(NF, CQ):
            x_rd[c].wait_recv()
            out_ref[pl.ds(off_m + c * rows, rows), :] = (
                send_buf[c, :, :] + comm_x[c, :, :]
            )

        for c in range(HD):
            yr_rd[c].wait_recv()
            fold(off_d, c, comm_d, c)
        for c in range(HD):
            zr_rd[c].wait_recv()
            fold(off_d, HD + c, comm_d, HD + c)

        for r in x_rd + yf_rd + zf_rd + yr_rd + zr_rd:
            r.wait_send()

    return pl.pallas_call(
        body,
        out_shape=jax.ShapeDtypeStruct((m, n), jnp.bfloat16),
        in_specs=[pl.BlockSpec(memory_space=pltpu.VMEM)],
        out_specs=pl.BlockSpec(memory_space=pltpu.VMEM),
        scratch_shapes=[
            pltpu.VMEM((NX, rows, n), jnp.bfloat16),
            pltpu.VMEM((NX, rows, n), jnp.bfloat16),
            pltpu.VMEM((NF, rows, n), jnp.bfloat16),
            pltpu.VMEM((NF, rows, n), jnp.bfloat16),
            pltpu.VMEM((ND, rows, n), jnp.bfloat16),
            pltpu.SemaphoreType.DMA((NX,)),
            pltpu.SemaphoreType.DMA((NX,)),
            pltpu.SemaphoreType.DMA((NF,)),
            pltpu.SemaphoreType.DMA((NF,)),
            pltpu.SemaphoreType.DMA((NF,)),
            pltpu.SemaphoreType.DMA((NF,)),
            pltpu.SemaphoreType.DMA((HD,)),
            pltpu.SemaphoreType.DMA((HD,)),
            pltpu.SemaphoreType.DMA((HD,)),
            pltpu.SemaphoreType.DMA((HD,)),
        ],
        compiler_params=pltpu.CompilerParams(collective_id=0),
    )(x)
